# baseline (speedup 1.0000x reference)
import functools

import jax
import jax.numpy as jnp
from jax import lax
from jax.experimental import pallas as pl
from jax.experimental.pallas import tpu as pltpu
from jax.experimental.pallas import tpu_sc as plsc

B = 1024
V = 100000
VS = 38400
VT = V - VS
BC = 800
NC = 2
NS = 16
NW = NC * NS
L = 16
NI = B // 128
WSPAN = VS // NW
WJC = 40
NCHK = WSPAN // WJC
NBUF = 2
NG = B // L
FPAD = 64
NEG_INF = float("-inf")


def _build(interpret=False):
  mesh = plsc.VectorSubcoreMesh(
      core_axis_name="c", subcore_axis_name="s", num_cores=NC, num_subcores=NS
  )

  @functools.partial(
    pl.kernel,
    out_type=(
        jax.ShapeDtypeStruct((NW, B), jnp.float32),
        jax.ShapeDtypeStruct((NW, B), jnp.float32),
    ),
    mesh=mesh,
    interpret=interpret,
    scratch_types=[
        [pltpu.VMEM((WJC, B), jnp.float32) for _ in range(NBUF)],
        pltpu.VMEM((B,), jnp.int32),
        pltpu.VMEM((B,), jnp.float32),
        pltpu.VMEM((B,), jnp.float32),
        pltpu.VMEM((FPAD,), jnp.int32),
        [pltpu.SemaphoreType.DMA] * NBUF,
    ],
    compiler_params=pltpu.CompilerParams(
        needs_layout_passes=False, use_tc_tiling_on_sc=True
    ),
  )
  def _cw_loss_sc(xt_hbm, y_hbm, outm_hbm, outt_hbm, buf, yv, accb, tvb,
                  flags, sems):
      c = lax.axis_index("c")
      s = lax.axis_index("s")
      w = s * NC + c
      col0 = VT + w * WSPAN

      pltpu.sync_copy(y_hbm, yv)

      lane = lax.iota(jnp.int32, L)
      neg = jnp.full((L,), NEG_INF, jnp.float32)
      zero = jnp.zeros((L,), jnp.int32)
      one = jnp.full((L,), 1, jnp.int32)

      def chunk_src(ch):
          return xt_hbm.at[pl.ds(col0 + ch * WJC, WJC), pl.ds(0, B)]

      for ch in range(NBUF):
          pltpu.async_copy(chunk_src(ch), buf[ch], sems[ch])

      for k in range(FPAD // L):
          flags[pl.ds(k * L, L)] = zero
      for k in range(NG):
          rel = yv[pl.ds(k * L, L)] - col0
          inw = (rel >= 0) & (rel < WSPAN)
          chix = jnp.clip(rel, 0, WSPAN - 1) // WJC
          plsc.store_scatter(flags, [chix], one, mask=inw)

      for k in range(NG):
          accb[pl.ds(k * L, L)] = neg
          tvb[pl.ds(k * L, L)] = neg

      def process(ch, b):
          j0c = col0 + ch * WJC

          fvec = flags[pl.ds(ch, L)]

          @pl.when(fvec[0] != 0)
          def _():
              for k in range(NG):
                  ycol = yv[pl.ds(k * L, L)] - j0c
                  inb = (ycol >= 0) & (ycol < WJC)
                  idxj = jnp.clip(ycol, 0, WJC - 1)
                  idxi = k * L + lane
                  g = plsc.load_gather(buf[b], [idxj, idxi], mask=inb)
                  tvb[pl.ds(k * L, L)] = jnp.where(
                      inb, g, tvb[pl.ds(k * L, L)]
                  )
                  plsc.store_scatter(buf[b], [idxj, idxi], neg, mask=inb)

          def itile_body(it, _):
              ioff = it * 128

              def tile_body(jt, accs):
                  out = list(accs)
                  for jj in range(8):
                      for k in range(8):
                          out[k] = jnp.maximum(
                              out[k],
                              buf[b][jt * 8 + jj, pl.ds(ioff + k * L, L)],
                          )
                  return tuple(out)

              accs = lax.fori_loop(
                  0, WJC // 8, tile_body,
                  tuple(accb[pl.ds(ioff + k * L, L)] for k in range(8)),
              )
              for k in range(8):
                  accb[pl.ds(ioff + k * L, L)] = accs[k]
              return 0

          lax.fori_loop(0, NI, itile_body, 0)

      def loop_body(it, _):
          for b in range(NBUF):
              ch = it * NBUF + b
              pltpu.make_async_copy(chunk_src(ch), buf[b], sems[b]).wait()
              process(ch, b)

              @pl.when(ch + NBUF < NCHK)
              def _():
                  pltpu.async_copy(chunk_src(ch + NBUF), buf[b], sems[b])

          return 0

      nfull = (NCHK // NBUF) * NBUF
      lax.fori_loop(0, NCHK // NBUF, loop_body, 0)
      for ch in range(nfull, NCHK):
          b = ch % NBUF
          pltpu.make_async_copy(chunk_src(ch), buf[b], sems[b]).wait()
          process(ch, b)

      pltpu.sync_copy(accb, outm_hbm.at[w])
      pltpu.sync_copy(tvb, outt_hbm.at[w])

  return _cw_loss_sc


_impl = _build()


def _tc_body(xb, yb, mo, to):
    i = pl.program_id(0)
    col0 = i * BC
    colid = lax.broadcasted_iota(jnp.int32, (BC, 1), 0) + col0
    mask = colid == yb[...]
    xv = xb[...]
    m = jnp.max(jnp.where(mask, NEG_INF, xv), axis=0)
    t = jnp.max(jnp.where(mask, xv, NEG_INF), axis=0)

    @pl.when(i == 0)
    def _():
        mo[...] = jnp.full((B,), NEG_INF, jnp.float32)
        to[...] = jnp.full((B,), NEG_INF, jnp.float32)

    mo[...] = jnp.maximum(mo[...], m)
    to[...] = jnp.maximum(to[...], t)


_tc_partial = pl.pallas_call(
    _tc_body,
    grid=(VT // BC,),
    in_specs=[
        pl.BlockSpec((BC, B), lambda i: (i, 0)),
        pl.BlockSpec((B,), lambda i: (0,)),
    ],
    out_specs=(
        pl.BlockSpec((B,), lambda i: (0,)),
        pl.BlockSpec((B,), lambda i: (0,)),
    ),
    out_shape=(
        jax.ShapeDtypeStruct((B,), jnp.float32),
        jax.ShapeDtypeStruct((B,), jnp.float32),
    ),
)


def _comb_body(ms, ts, m2, t2, o):
    m = jnp.maximum(jnp.max(ms[...], axis=0), m2[...])
    t = jnp.maximum(jnp.max(ts[...], axis=0), t2[...])
    o[...] = m - t


_combine = pl.pallas_call(
    _comb_body,
    out_shape=jax.ShapeDtypeStruct((B,), jnp.float32),
)


def kernel(x, y):
    xt = x.T
    y32 = y.astype(jnp.int32)
    ms, ts = _impl(xt, y32)
    mt, tt = _tc_partial(xt, y32)
    return _combine(ms, ts, mt, tt)


# speedup vs baseline: 1.0453x; 1.0453x over previous
import functools

import jax
import jax.numpy as jnp
from jax import lax
from jax.experimental import pallas as pl
from jax.experimental.pallas import tpu as pltpu
from jax.experimental.pallas import tpu_sc as plsc

B = 1024
V = 100000
VS = 44800
VT = V - VS
BC = 800
NC = 2
NS = 16
NW = NC * NS
L = 16
NI = B // 128
WSPAN = VS // NW
WJC = 40
NCHK = WSPAN // WJC
NBUF = 2
NG = B // L
FPAD = 64
NEG_INF = float("-inf")


def _build(interpret=False):
  mesh = plsc.VectorSubcoreMesh(
      core_axis_name="c", subcore_axis_name="s", num_cores=NC, num_subcores=NS
  )

  @functools.partial(
    pl.kernel,
    out_type=(
        jax.ShapeDtypeStruct((NW, B), jnp.float32),
        jax.ShapeDtypeStruct((NW, B), jnp.float32),
    ),
    mesh=mesh,
    interpret=interpret,
    scratch_types=[
        [pltpu.VMEM((WJC, B), jnp.float32) for _ in range(NBUF)],
        pltpu.VMEM((B,), jnp.int32),
        pltpu.VMEM((B,), jnp.float32),
        pltpu.VMEM((B,), jnp.float32),
        pltpu.VMEM((FPAD,), jnp.int32),
        [pltpu.SemaphoreType.DMA] * NBUF,
    ],
    compiler_params=pltpu.CompilerParams(
        needs_layout_passes=False, use_tc_tiling_on_sc=True
    ),
  )
  def _cw_loss_sc(xt_hbm, y_hbm, outm_hbm, outt_hbm, buf, yv, accb, tvb,
                  flags, sems):
      c = lax.axis_index("c")
      s = lax.axis_index("s")
      w = s * NC + c
      col0 = VT + w * WSPAN

      pltpu.sync_copy(y_hbm, yv)

      lane = lax.iota(jnp.int32, L)
      neg = jnp.full((L,), NEG_INF, jnp.float32)
      zero = jnp.zeros((L,), jnp.int32)
      one = jnp.full((L,), 1, jnp.int32)

      def chunk_src(ch):
          return xt_hbm.at[pl.ds(col0 + ch * WJC, WJC), pl.ds(0, B)]

      for ch in range(NBUF):
          pltpu.async_copy(chunk_src(ch), buf[ch], sems[ch])

      for k in range(FPAD // L):
          flags[pl.ds(k * L, L)] = zero
      for k in range(NG):
          rel = yv[pl.ds(k * L, L)] - col0
          inw = (rel >= 0) & (rel < WSPAN)
          chix = jnp.clip(rel, 0, WSPAN - 1) // WJC
          plsc.store_scatter(flags, [chix], one, mask=inw)

      for k in range(NG):
          accb[pl.ds(k * L, L)] = neg
          tvb[pl.ds(k * L, L)] = neg

      def process(ch, b):
          j0c = col0 + ch * WJC

          fvec = flags[pl.ds(ch, L)]

          @pl.when(fvec[0] != 0)
          def _():
              for k in range(NG):
                  ycol = yv[pl.ds(k * L, L)] - j0c
                  inb = (ycol >= 0) & (ycol < WJC)
                  idxj = jnp.clip(ycol, 0, WJC - 1)
                  idxi = k * L + lane
                  g = plsc.load_gather(buf[b], [idxj, idxi], mask=inb)
                  tvb[pl.ds(k * L, L)] = jnp.where(
                      inb, g, tvb[pl.ds(k * L, L)]
                  )
                  plsc.store_scatter(buf[b], [idxj, idxi], neg, mask=inb)

          def itile_body(it, _):
              ioff = it * 128

              def tile_body(jt, accs):
                  out = list(accs)
                  for jj in range(8):
                      for k in range(8):
                          out[k] = jnp.maximum(
                              out[k],
                              buf[b][jt * 8 + jj, pl.ds(ioff + k * L, L)],
                          )
                  return tuple(out)

              accs = lax.fori_loop(
                  0, WJC // 8, tile_body,
                  tuple(accb[pl.ds(ioff + k * L, L)] for k in range(8)),
              )
              for k in range(8):
                  accb[pl.ds(ioff + k * L, L)] = accs[k]
              return 0

          lax.fori_loop(0, NI, itile_body, 0)

      def loop_body(it, _):
          for b in range(NBUF):
              ch = it * NBUF + b
              pltpu.make_async_copy(chunk_src(ch), buf[b], sems[b]).wait()
              process(ch, b)

              @pl.when(ch + NBUF < NCHK)
              def _():
                  pltpu.async_copy(chunk_src(ch + NBUF), buf[b], sems[b])

          return 0

      nfull = (NCHK // NBUF) * NBUF
      lax.fori_loop(0, NCHK // NBUF, loop_body, 0)
      for ch in range(nfull, NCHK):
          b = ch % NBUF
          pltpu.make_async_copy(chunk_src(ch), buf[b], sems[b]).wait()
          process(ch, b)

      pltpu.sync_copy(accb, outm_hbm.at[w])
      pltpu.sync_copy(tvb, outt_hbm.at[w])

  return _cw_loss_sc


_impl = _build()


def _tc_body(xb, yb, mo, to):
    i = pl.program_id(0)
    col0 = i * BC
    colid = lax.broadcasted_iota(jnp.int32, (BC, 1), 0) + col0
    mask = colid == yb[...]
    xv = xb[...]
    m = jnp.max(jnp.where(mask, NEG_INF, xv), axis=0)
    t = jnp.max(jnp.where(mask, xv, NEG_INF), axis=0)

    @pl.when(i == 0)
    def _():
        mo[...] = jnp.full((B,), NEG_INF, jnp.float32)
        to[...] = jnp.full((B,), NEG_INF, jnp.float32)

    mo[...] = jnp.maximum(mo[...], m)
    to[...] = jnp.maximum(to[...], t)


_tc_partial = pl.pallas_call(
    _tc_body,
    grid=(VT // BC,),
    in_specs=[
        pl.BlockSpec((BC, B), lambda i: (i, 0)),
        pl.BlockSpec((B,), lambda i: (0,)),
    ],
    out_specs=(
        pl.BlockSpec((B,), lambda i: (0,)),
        pl.BlockSpec((B,), lambda i: (0,)),
    ),
    out_shape=(
        jax.ShapeDtypeStruct((B,), jnp.float32),
        jax.ShapeDtypeStruct((B,), jnp.float32),
    ),
)


def _comb_body(ms, ts, m2, t2, o):
    m = jnp.maximum(jnp.max(ms[...], axis=0), m2[...])
    t = jnp.maximum(jnp.max(ts[...], axis=0), t2[...])
    o[...] = m - t


_combine = pl.pallas_call(
    _comb_body,
    out_shape=jax.ShapeDtypeStruct((B,), jnp.float32),
)


def kernel(x, y):
    xt = x.T
    y32 = y.astype(jnp.int32)
    ms, ts = _impl(xt, y32)
    mt, tt = _tc_partial(xt, y32)
    return _combine(ms, ts, mt, tt)
